# Initial kernel scaffold; baseline (speedup 1.0000x reference)
import jax
import jax.numpy as jnp
from jax.experimental import pallas as pl


def kernel(x):
    raise NotImplementedError("write your pallas kernel here")


# speedup vs baseline: 1.0587x; 1.0587x over previous
import jax
import jax.numpy as jnp
from jax import lax
from jax.experimental import pallas as pl

D_MODEL = 2048
BLOCK_T = 512


def _onehot_block(idx_ref, out_ref):
    idx = idx_ref[0, 0, :]
    iota = lax.broadcasted_iota(jnp.int32, (BLOCK_T, D_MODEL), 1)
    out_ref[...] = (iota == idx[:, None]).astype(jnp.float32)


def kernel(x):
    b, s, _ = x.shape
    n_tok = b * s
    grid = n_tok // BLOCK_T
    idx3 = x.reshape(grid, 1, BLOCK_T)
    out = pl.pallas_call(
        _onehot_block,
        grid=(grid,),
        in_specs=[pl.BlockSpec((1, 1, BLOCK_T), lambda i: (i, 0, 0))],
        out_specs=pl.BlockSpec((BLOCK_T, D_MODEL), lambda i: (i, 0)),
        out_shape=jax.ShapeDtypeStruct((n_tok, D_MODEL), jnp.float32),
    )(idx3)
    return (out.reshape(b, s, D_MODEL),)
